# baseline (speedup 1.0000x reference)
import functools

import jax
import jax.numpy as jnp
from jax.experimental import pallas as pl
from jax.experimental.pallas import tpu as pltpu
from jax.experimental.pallas import tpu_sc as plsc

_NSEL = 300
_TW = 128
_BOX0 = 96


def _pack_body(logit_ref, box_ref, rm_ref, tab_ref):
    n = logit_ref.shape[1]
    p = jax.nn.sigmoid(logit_ref[0])
    rm_ref[0, 0, :] = jnp.max(p, axis=-1)
    bx = box_ref[0]
    xyxy = jnp.concatenate(
        [bx[:, :2] - 0.5 * bx[:, 2:], bx[:, :2] + 0.5 * bx[:, 2:]], axis=1)
    c = p.shape[1]
    tab_ref[...] = jnp.concatenate(
        [p, jnp.zeros((n, _BOX0 - c), p.dtype), xyxy,
         jnp.zeros((n, _TW - _BOX0 - 4), p.dtype)], axis=1)


def _pack_and_rowmax(pred_logits, pred_boxes, chunk=2000):
    B, N, C = pred_logits.shape
    g = N // chunk
    rm, tab = pl.pallas_call(
        _pack_body,
        grid=(B, g),
        in_specs=[
            pl.BlockSpec((1, chunk, C), lambda b, j: (b, j, 0)),
            pl.BlockSpec((1, chunk, 4), lambda b, j: (b, j, 0)),
        ],
        out_specs=[
            pl.BlockSpec((1, 1, chunk), lambda b, j: (b * g + j, 0, 0)),
            pl.BlockSpec((chunk, _TW), lambda b, j: (b * g + j, 0)),
        ],
        out_shape=[
            jax.ShapeDtypeStruct((B * g, 1, chunk), pred_logits.dtype),
            jax.ShapeDtypeStruct((B * N, _TW), pred_logits.dtype),
        ],
    )(pred_logits, pred_boxes)
    return rm.reshape(B, N), tab


def _sc_gather(table, idx):
    info = plsc.get_sparse_core_info()
    nc, ns = info.num_cores, info.num_subcores
    r = idx.shape[0] // (nc * ns)

    @functools.partial(
        pl.kernel,
        out_type=jax.ShapeDtypeStruct((idx.shape[0], _TW), jnp.float32),
        mesh=plsc.VectorSubcoreMesh(core_axis_name="c", subcore_axis_name="s"),
        scratch_types=[
            pltpu.VMEM((r,), jnp.int32),
            pltpu.VMEM((r, _TW), jnp.float32),
            pltpu.SemaphoreType.DMA,
        ],
    )
    def gk(tab_hbm, idx_hbm, out, idx_v, rows_v, sem):
        wid = jax.lax.axis_index("s") * nc + jax.lax.axis_index("c")
        base = wid * r
        pltpu.sync_copy(idx_hbm.at[pl.ds(base, r)], idx_v)
        pltpu.async_copy(tab_hbm.at[idx_v], rows_v, sem).wait()
        pltpu.sync_copy(rows_v, out.at[pl.ds(base, r)])

    return gk(table, idx)


def kernel(pred_logits, pred_boxes, target_sizes):
    B, N, C = pred_logits.shape
    row_max, table = _pack_and_rowmax(pred_logits, pred_boxes)

    _, rows = jax.lax.top_k(row_max, _NSEL)
    rows = jnp.sort(rows, axis=-1)

    gidx = (rows + N * jnp.arange(B, dtype=rows.dtype)[:, None]).reshape(-1)
    rem = gidx.shape[0] % (8 * 32)
    if rem:
        gidx = jnp.concatenate(
            [gidx, jnp.zeros((8 * 32 - rem,), gidx.dtype)])
    g = _sc_gather(table, gidx)[: B * _NSEL].reshape(B, _NSEL, _TW)

    cand = g[:, :, :C].reshape(B, _NSEL * C)
    scores, pos = jax.lax.top_k(cand, _NSEL)
    labels = pos % C
    slot = pos // C

    xyxy = jnp.take_along_axis(
        g[:, :, _BOX0:_BOX0 + 4], slot[:, :, None], axis=1)
    img_h = target_sizes[:, 0].astype(jnp.float32)
    img_w = target_sizes[:, 1].astype(jnp.float32)
    scale = jnp.stack([img_w, img_h, img_w, img_h], axis=1)
    boxes = xyxy * scale[:, None, :]
    return boxes, scores, labels


# speedup vs baseline: 1.7271x; 1.7271x over previous
import functools

import jax
import jax.numpy as jnp
from jax.experimental import pallas as pl
from jax.experimental.pallas import tpu as pltpu
from jax.experimental.pallas import tpu_sc as plsc

_NSEL = 300
_TW = 128
_BOX0 = 96


def _pack_body(logit_ref, box_ref, rm_ref, tab_ref):
    n = logit_ref.shape[1]
    p = jax.nn.sigmoid(logit_ref[0])
    rm_ref[0, 0, :] = jnp.max(p, axis=-1)
    bx = box_ref[0]
    xyxy = jnp.concatenate(
        [bx[:, :2] - 0.5 * bx[:, 2:], bx[:, :2] + 0.5 * bx[:, 2:]], axis=1)
    c = p.shape[1]
    tab_ref[...] = jnp.concatenate(
        [p, jnp.zeros((n, _BOX0 - c), p.dtype), xyxy,
         jnp.zeros((n, _TW - _BOX0 - 4), p.dtype)], axis=1)


def _pack_and_rowmax(pred_logits, pred_boxes, chunk=2000):
    B, N, C = pred_logits.shape
    g = N // chunk
    rm, tab = pl.pallas_call(
        _pack_body,
        grid=(B, g),
        in_specs=[
            pl.BlockSpec((1, chunk, C), lambda b, j: (b, j, 0)),
            pl.BlockSpec((1, chunk, 4), lambda b, j: (b, j, 0)),
        ],
        out_specs=[
            pl.BlockSpec((1, 1, chunk), lambda b, j: (b * g + j, 0, 0)),
            pl.BlockSpec((chunk, _TW), lambda b, j: (b * g + j, 0)),
        ],
        out_shape=[
            jax.ShapeDtypeStruct((B * g, 1, chunk), pred_logits.dtype),
            jax.ShapeDtypeStruct((B * N, _TW), pred_logits.dtype),
        ],
    )(pred_logits, pred_boxes)
    return rm.reshape(B, N), tab


def _sc_gather(table, idx):
    info = plsc.get_sparse_core_info()
    nc, ns = info.num_cores, info.num_subcores
    r = idx.shape[0] // (nc * ns)

    @functools.partial(
        pl.kernel,
        out_type=jax.ShapeDtypeStruct((idx.shape[0], _TW), jnp.float32),
        mesh=plsc.VectorSubcoreMesh(core_axis_name="c", subcore_axis_name="s"),
        scratch_types=[
            pltpu.VMEM((r,), jnp.int32),
            pltpu.VMEM((r, _TW), jnp.float32),
            pltpu.SemaphoreType.DMA,
        ],
    )
    def gk(tab_hbm, idx_hbm, out, idx_v, rows_v, sem):
        wid = jax.lax.axis_index("s") * nc + jax.lax.axis_index("c")
        base = wid * r
        pltpu.sync_copy(idx_hbm.at[pl.ds(base, r)], idx_v)
        pltpu.async_copy(tab_hbm.at[idx_v], rows_v, sem).wait()
        pltpu.sync_copy(rows_v, out.at[pl.ds(base, r)])

    return gk(table, idx)


def kernel(pred_logits, pred_boxes, target_sizes):
    B, N, C = pred_logits.shape
    row_max, table = _pack_and_rowmax(pred_logits, pred_boxes)

    g1 = row_max.reshape(B, N // 10, 10)
    _, grp = jax.lax.top_k(g1.max(axis=-1), _NSEL)
    grp = jnp.sort(grp, axis=-1)
    cand_rm = jnp.take_along_axis(g1, grp[:, :, None], axis=1).reshape(B, _NSEL * 10)
    _, p1 = jax.lax.top_k(cand_rm, _NSEL)
    rows = jnp.take_along_axis(grp, p1 // 10, axis=1) * 10 + p1 % 10
    rows = jnp.sort(rows, axis=-1)

    gidx = (rows + N * jnp.arange(B, dtype=rows.dtype)[:, None]).reshape(-1)
    rem = gidx.shape[0] % (8 * 32)
    if rem:
        gidx = jnp.concatenate(
            [gidx, jnp.zeros((8 * 32 - rem,), gidx.dtype)])
    g = _sc_gather(table, gidx)[: B * _NSEL].reshape(B, _NSEL, _TW)

    cand = g[:, :, :C].reshape(B, _NSEL * C)
    g2 = cand.reshape(B, _NSEL * C // 13, 13)
    _, blk = jax.lax.top_k(g2.max(axis=-1), _NSEL)
    blk = jnp.sort(blk, axis=-1)
    cand2 = jnp.take_along_axis(g2, blk[:, :, None], axis=1).reshape(B, _NSEL * 13)
    scores, p2 = jax.lax.top_k(cand2, _NSEL)
    pos = jnp.take_along_axis(blk, p2 // 13, axis=1) * 13 + p2 % 13
    labels = pos % C
    slot = pos // C

    xyxy = jnp.take_along_axis(
        g[:, :, _BOX0:_BOX0 + 4], slot[:, :, None], axis=1)
    img_h = target_sizes[:, 0].astype(jnp.float32)
    img_w = target_sizes[:, 1].astype(jnp.float32)
    scale = jnp.stack([img_w, img_h, img_w, img_h], axis=1)
    boxes = xyxy * scale[:, None, :]
    return boxes, scores, labels
